# baseline (speedup 1.0000x reference)
import functools

import jax
import jax.numpy as jnp
from jax import lax
from jax.experimental import pallas as pl
from jax.experimental.pallas import tpu as pltpu
from jax.experimental.pallas import tpu_sc as plsc

_N = 10000
_E = 320000
_G = 64
_NC = 2
_NS = 16
_K = 64
_EP = 327680
_NCH1 = _EP // (_NC * _NS * _K)
_NCH2 = _EP // (_NS * _K)
_GRP = 40
_NBUF = 4
_NDUMP = 400
_NACC = _N + _NDUMP
_WC = 400
_NWC = _N // _WC
_ZR = 16
_DH = 128


@functools.cache
def _mesh():
    return plsc.VectorSubcoreMesh(core_axis_name="c", subcore_axis_name="s",
                                  num_cores=_NC, num_subcores=_NS)


def _zero_acc(zb_v, acc, s, D, sem):
    @pl.loop(0, _ZR)
    def _(r):
        @pl.loop(0, D // 16)
        def _(j):
            zb_v[r, pl.ds(j * 16, 16)] = jnp.zeros((16,), jnp.float32)

    for r in range(2):
        j = s + _NS * r

        @pl.when(j < _NWC)
        def _():
            @pl.loop(0, _WC // _ZR)
            def _(m):
                pltpu.async_copy(
                    zb_v, acc.at[pl.ds(j * _WC + m * _ZR, _ZR)], sem)

            @pl.loop(0, _WC // _ZR)
            def _(m):
                pltpu.make_async_copy(
                    zb_v, acc.at[pl.ds(j * _WC, _ZR)], sem).wait()


def _write_out(acc, dst_ref, s, sem):
    for r in range(2):
        j = s + _NS * r

        @pl.when(j < _NWC)
        def _():
            pltpu.async_copy(acc.at[pl.ds(j * _WC, _WC)],
                             dst_ref.at[pl.ds(j * _WC, _WC)], sem)

    for r in range(2):
        j = s + _NS * r

        @pl.when(j < _NWC)
        def _():
            pltpu.make_async_copy(acc.at[pl.ds(j * _WC, _WC)],
                                  dst_ref.at[pl.ds(j * _WC, _WC)], sem).wait()


def _agg_pipeline(idx_src, idx_dst, h_src, rows, acc,
                  src_v, dst_v, gsem, ssem, n_groups):
    @pl.loop(0, n_groups)
    def _(g):
        pltpu.sync_copy(idx_src.at[g], src_v)
        pltpu.sync_copy(idx_dst.at[g], dst_v)
        for b in range(_NBUF):
            pltpu.async_copy(h_src.at[src_v.at[b]], rows[b], gsem.at[b])

        @pl.loop(0, _GRP // _NBUF)
        def _(q):
            for b in range(_NBUF):
                ch = q * _NBUF + b
                pltpu.make_async_copy(
                    h_src.at[src_v.at[ch]], rows[b], gsem.at[b]).wait()
                pltpu.async_copy(
                    rows[b], acc.at[dst_v.at[ch]], ssem.at[b], add=True)
                nxt = ch + _NBUF

                @pl.when(nxt < _GRP)
                def _():
                    pltpu.make_async_copy(
                        rows[b], acc.at[dst_v.at[ch]], ssem.at[b]).wait()
                    pltpu.async_copy(
                        h_src.at[src_v.at[nxt]], rows[b], gsem.at[b])

        for b in range(_NBUF):
            pltpu.make_async_copy(
                rows[b], acc.at[dst_v.at[0]], ssem.at[b]).wait()


def _sc_degree(dst_t):
    @functools.partial(
        pl.kernel,
        out_type=jax.ShapeDtypeStruct((_NC, _N, _DH), jnp.float32),
        mesh=_mesh(),
        scratch_types=[
            pltpu.VMEM((_GRP, _K), jnp.int32),
            pltpu.VMEM((_K, _DH), jnp.float32),
            pltpu.VMEM((_ZR, _DH), jnp.float32),
            pltpu.SemaphoreType.DMA,
            pltpu.SemaphoreType.DMA,
            pltpu.VMEM_SHARED((_NACC, _DH), jnp.float32),
        ],
    )
    def deg_kernel(dst_hbm, out_hbm, dst_v, ones_v, zb_v, ssem, msem, acc):
        c = lax.axis_index("c")
        s = lax.axis_index("s")

        @pl.loop(0, _K)
        def _(r):
            @pl.loop(0, _DH // 16)
            def _(j):
                ones_v[r, pl.ds(j * 16, 16)] = jnp.ones((16,), jnp.float32)

        _zero_acc(zb_v, acc, s, _DH, msem)
        plsc.subcore_barrier()

        @pl.loop(0, _NCH1 // _GRP)
        def _(g):
            pltpu.sync_copy(dst_hbm.at[c, s, g], dst_v)

            @pl.loop(0, _GRP)
            def _(ch):
                pltpu.async_copy(ones_v, acc.at[dst_v.at[ch]], ssem, add=True)

            @pl.loop(0, _GRP)
            def _(ch):
                pltpu.make_async_copy(ones_v, acc.at[dst_v.at[0]], ssem).wait()

        plsc.subcore_barrier()
        _write_out(acc, out_hbm.at[c], s, msem)

    return deg_kernel(dst_t)


def _agg_scratch(D):
    return [
        pltpu.VMEM((_GRP, _K), jnp.int32),
        pltpu.VMEM((_GRP, _K), jnp.int32),
    ] + [pltpu.VMEM((_K, D), jnp.float32) for _ in range(_NBUF)] + [
        pltpu.VMEM((_ZR, D), jnp.float32),
        pltpu.SemaphoreType.DMA((_NBUF,)),
        pltpu.SemaphoreType.DMA((_NBUF,)),
        pltpu.SemaphoreType.DMA,
        pltpu.VMEM_SHARED((_NACC, D), jnp.float32),
    ]


def _sc_agg_l1(h, src_t, dst_t):
    D = h.shape[1]

    @functools.partial(
        pl.kernel,
        out_type=jax.ShapeDtypeStruct((_NC, _N, D), jnp.float32),
        mesh=_mesh(),
        scratch_types=_agg_scratch(D),
    )
    def agg_kernel(h_hbm, src_hbm, dst_hbm, out_hbm,
                   src_v, dst_v, r0, r1, r2, r3, zb_v, gsem, ssem, msem, acc):
        c = lax.axis_index("c")
        s = lax.axis_index("s")
        _zero_acc(zb_v, acc, s, D, msem)
        plsc.subcore_barrier()
        _agg_pipeline(src_hbm.at[c, s], dst_hbm.at[c, s], h_hbm,
                      [r0, r1, r2, r3], acc, src_v, dst_v, gsem, ssem,
                      _NCH1 // _GRP)
        plsc.subcore_barrier()
        _write_out(acc, out_hbm.at[c], s, msem)

    return agg_kernel(h, src_t, dst_t)


def _sc_agg_cols(h2, src_t, dst_t):
    @functools.partial(
        pl.kernel,
        out_type=jax.ShapeDtypeStruct((_NC, _N, _DH), jnp.float32),
        mesh=_mesh(),
        scratch_types=_agg_scratch(_DH),
    )
    def agg_kernel(h_hbm, src_hbm, dst_hbm, out_hbm,
                   src_v, dst_v, r0, r1, r2, r3, zb_v, gsem, ssem, msem, acc):
        c = lax.axis_index("c")
        s = lax.axis_index("s")
        _zero_acc(zb_v, acc, s, _DH, msem)
        plsc.subcore_barrier()
        _agg_pipeline(src_hbm.at[s], dst_hbm.at[s], h_hbm.at[c],
                      [r0, r1, r2, r3], acc, src_v, dst_v, gsem, ssem,
                      _NCH2 // _GRP)
        plsc.subcore_barrier()
        _write_out(acc, out_hbm.at[c], s, msem)

    return agg_kernel(h2, src_t, dst_t)


_TCR = 2000


def _tc_scale(degp, x):
    F = x.shape[1]

    def body(dp_ref, x_ref, dinv_ref, xp_ref):
        dp = dp_ref[...]
        deg = dp[0, :, 0:1] + dp[1, :, 0:1] + 1.0
        dinv = lax.rsqrt(deg)
        dinv_ref[...] = dinv
        xp_ref[...] = x_ref[...] * dinv

    return pl.pallas_call(
        body,
        grid=(_N // _TCR,),
        in_specs=[pl.BlockSpec((2, _TCR, _DH), lambda i: (0, i, 0)),
                  pl.BlockSpec((_TCR, F), lambda i: (i, 0))],
        out_specs=[pl.BlockSpec((_TCR, 1), lambda i: (i, 0)),
                   pl.BlockSpec((_TCR, F), lambda i: (i, 0))],
        out_shape=[jax.ShapeDtypeStruct((_N, 1), jnp.float32),
                   jax.ShapeDtypeStruct((_N, F), jnp.float32)],
    )(degp, x)


def _split_cols(out_ref, t):
    out_ref[0] = t[:, :_DH]
    out_ref[1] = t[:, _DH:]


def _tc_layer1(p, xp, dinv, w1p, b1p, w2p):
    F = xp.shape[1]

    def body(p_ref, xp_ref, dinv_ref, w1_ref, b1_ref, w2_ref, out_ref):
        pp = p_ref[...]
        dinv = dinv_ref[...]
        g = dinv * (pp[0] + pp[1] + xp_ref[...])
        t = jnp.dot(g, w1_ref[...], preferred_element_type=jnp.float32)
        t = jnp.maximum(t + b1_ref[...], 0.0)
        t = dinv * jnp.dot(t, w2_ref[...], preferred_element_type=jnp.float32)
        _split_cols(out_ref, t)

    return pl.pallas_call(
        body,
        grid=(_N // _TCR,),
        in_specs=[pl.BlockSpec((2, _TCR, F), lambda i: (0, i, 0)),
                  pl.BlockSpec((_TCR, F), lambda i: (i, 0)),
                  pl.BlockSpec((_TCR, 1), lambda i: (i, 0)),
                  pl.BlockSpec(w1p.shape, lambda i: (0, 0)),
                  pl.BlockSpec(b1p.shape, lambda i: (0, 0)),
                  pl.BlockSpec(w2p.shape, lambda i: (0, 0))],
        out_specs=pl.BlockSpec((2, _TCR, _DH), lambda i: (0, i, 0)),
        out_shape=jax.ShapeDtypeStruct((_NC, _N, _DH), jnp.float32),
    )(p, xp, dinv, w1p, b1p, w2p)


def _merge_halves(p_ref, hp_ref):
    pp = p_ref[...]
    hh = hp_ref[...]
    return jnp.concatenate([pp[0] + hh[0], pp[1] + hh[1]], axis=1)


def _tc_mid(p, hp, dinv, bp, wp):
    def body(p_ref, hp_ref, dinv_ref, b_ref, w_ref, out_ref):
        dinv = dinv_ref[...]
        g = _merge_halves(p_ref, hp_ref)
        a = jnp.maximum(dinv * g + b_ref[...], 0.0)
        t = dinv * jnp.dot(a, w_ref[...], preferred_element_type=jnp.float32)
        _split_cols(out_ref, t)

    return pl.pallas_call(
        body,
        grid=(_N // _TCR,),
        in_specs=[pl.BlockSpec((2, _TCR, _DH), lambda i: (0, i, 0)),
                  pl.BlockSpec((2, _TCR, _DH), lambda i: (0, i, 0)),
                  pl.BlockSpec((_TCR, 1), lambda i: (i, 0)),
                  pl.BlockSpec(bp.shape, lambda i: (0, 0)),
                  pl.BlockSpec(wp.shape, lambda i: (0, 0))],
        out_specs=pl.BlockSpec((2, _TCR, _DH), lambda i: (0, i, 0)),
        out_shape=jax.ShapeDtypeStruct((_NC, _N, _DH), jnp.float32),
    )(p, hp, dinv, bp, wp)


def _tc_last(p, hp, dinv, bp):
    D = 2 * _DH

    def body(p_ref, hp_ref, dinv_ref, b_ref, out_ref):
        dinv = dinv_ref[...]
        g = _merge_halves(p_ref, hp_ref)
        out_ref[...] = jnp.maximum(dinv * g + b_ref[...], 0.0)

    return pl.pallas_call(
        body,
        grid=(_N // _TCR,),
        in_specs=[pl.BlockSpec((2, _TCR, _DH), lambda i: (0, i, 0)),
                  pl.BlockSpec((2, _TCR, _DH), lambda i: (0, i, 0)),
                  pl.BlockSpec((_TCR, 1), lambda i: (i, 0)),
                  pl.BlockSpec(bp.shape, lambda i: (0, 0))],
        out_specs=pl.BlockSpec((_TCR, D), lambda i: (i, 0)),
        out_shape=jax.ShapeDtypeStruct((_N, D), jnp.float32),
    )(p, hp, dinv, bp)


def _tc_pool_head(a4, batch_row, fc1w, fc1b2, fc2w, fc2b2, hdim):
    D = a4.shape[1]
    NEG = -3.0e38

    def body(a4_ref, bt_ref, w1_ref, b1_ref, w2_ref, b2_ref, out_ref, max_s):
        a = a4_ref[...]
        btr = bt_ref[...]
        gi = lax.broadcasted_iota(jnp.int32, (_G, _N), 0)
        oh = (btr == gi).astype(jnp.float32)
        gsum = jnp.dot(oh, a, preferred_element_type=jnp.float32)
        cnt = jnp.sum(oh, axis=1, keepdims=True)

        bm = jnp.max(a.reshape(_N // 8, 8, D), axis=1)
        gidc = lax.broadcasted_iota(jnp.int32, (_G, 1), 0)
        ki = lax.broadcasted_iota(jnp.int32, (_N // 8, 1), 0)
        ri = lax.broadcasted_iota(jnp.int32, (8, 1), 0)

        def seg(g, carry):
            e0 = jnp.sum(jnp.where(gidc <= g, cnt, 0.0)).astype(jnp.int32)
            n0 = jnp.sum(jnp.where(gidc == g, cnt, 0.0)).astype(jnp.int32)
            s0 = e0 - n0
            kb0 = (s0 + 7) // 8
            kb1 = e0 // 8
            m = jnp.max(jnp.where((ki >= kb0) & (ki < kb1), bm, NEG),
                        axis=0, keepdims=True)
            hb = jnp.clip(s0 // 8, 0, _N // 8 - 1)
            tb = jnp.clip(kb1, 0, _N // 8 - 1)
            hrows = a4_ref[pl.ds(hb * 8, 8), :]
            hmask = (ri + hb * 8 >= s0) & (ri + hb * 8 < e0)
            m = jnp.maximum(m, jnp.max(jnp.where(hmask, hrows, NEG),
                                       axis=0, keepdims=True))
            trows = a4_ref[pl.ds(tb * 8, 8), :]
            tmask = (ri + tb * 8 >= s0) & (ri + tb * 8 < e0)
            m = jnp.maximum(m, jnp.max(jnp.where(tmask, trows, NEG),
                                       axis=0, keepdims=True))
            max_s[pl.ds(g, 1), :] = m
            return carry

        lax.fori_loop(0, _G, seg, 0)
        gmaxv = max_s[...]
        gmean = gsum / jnp.maximum(cnt, 1.0)
        gmax = jnp.where(cnt > 0.0, gmaxv, 0.0)
        z = jnp.concatenate([gmean[:, :hdim], gmax[:, :hdim]], axis=1)
        z = jnp.dot(z, w1_ref[...], preferred_element_type=jnp.float32)
        z = jnp.maximum(z + b1_ref[...], 0.0)
        logits = jnp.dot(z, w2_ref[...], preferred_element_type=jnp.float32)
        logits = logits + b2_ref[...]
        mx = jnp.max(logits, axis=1, keepdims=True)
        lse = jnp.log(jnp.sum(jnp.exp(logits - mx), axis=1, keepdims=True)) + mx
        out_ref[...] = logits - lse

    return pl.pallas_call(
        body,
        out_shape=jax.ShapeDtypeStruct((_G, fc2w.shape[1]), jnp.float32),
        scratch_shapes=[pltpu.VMEM((_G, D), jnp.float32)],
    )(a4, batch_row, fc1w, fc1b2, fc2w, fc2b2)


def kernel(x, edge_index, batch, W1, b1, W2, b2, W3, b3, W4, b4,
           fc1W, fc1b, fc2W, fc2b):
    H = W1.shape[1]
    D = 2 * _DH

    npad = _EP - _E
    d_src = (jnp.arange(npad, dtype=edge_index.dtype) * 61) % _N
    d_dst = _N + jnp.arange(npad, dtype=edge_index.dtype) % _NDUMP

    def _tiled(v, dv):
        nt = _NC * _NS
        return jnp.concatenate(
            [v.reshape(nt, _E // nt), dv.reshape(nt, npad // nt)],
            axis=1).reshape(-1)

    src_pad = _tiled(edge_index[0], d_src)
    dst_pad = _tiled(edge_index[1], d_dst)
    src1 = src_pad.reshape(_NC, _NS, _NCH1 // _GRP, _GRP, _K)
    dst1 = dst_pad.reshape(_NC, _NS, _NCH1 // _GRP, _GRP, _K)
    src2 = src_pad.reshape(_NS, _NCH2 // _GRP, _GRP, _K)
    dst2 = dst_pad.reshape(_NS, _NCH2 // _GRP, _GRP, _K)
    batch_row = batch.reshape(1, _N)

    pad_c = D - H
    w1p = jnp.pad(W1, ((0, 0), (0, pad_c)))
    w2p = jnp.pad(W2, ((0, pad_c), (0, pad_c)))
    w3p = jnp.pad(W3, ((0, pad_c), (0, pad_c)))
    w4p = jnp.pad(W4, ((0, pad_c), (0, pad_c)))
    b1p = jnp.pad(b1, (0, pad_c)).reshape(1, D)
    b2p = jnp.pad(b2, (0, pad_c)).reshape(1, D)
    b3p = jnp.pad(b3, (0, pad_c)).reshape(1, D)
    b4p = jnp.pad(b4, (0, pad_c)).reshape(1, D)
    fc1b2 = fc1b.reshape(1, -1)
    fc2b2 = fc2b.reshape(1, -1)

    degp = _sc_degree(dst1)
    dinv, xp = _tc_scale(degp, x)
    p1 = _sc_agg_l1(xp, src1, dst1)
    h2p = _tc_layer1(p1, xp, dinv, w1p, b1p, w2p)
    p2 = _sc_agg_cols(h2p, src2, dst2)
    h3p = _tc_mid(p2, h2p, dinv, b2p, w3p)
    p3 = _sc_agg_cols(h3p, src2, dst2)
    h4p = _tc_mid(p3, h3p, dinv, b3p, w4p)
    p4 = _sc_agg_cols(h4p, src2, dst2)
    a4 = _tc_last(p4, h4p, dinv, b4p)
    return _tc_pool_head(a4, batch_row, fc1W, fc1b2, fc2W, fc2b2, H)


# speedup vs baseline: 1.0600x; 1.0600x over previous
import dataclasses
import functools

import jax
import jax.numpy as jnp
from jax import lax
from jax.experimental import pallas as pl
from jax.experimental.pallas import tpu as pltpu
from jax.experimental.pallas import tpu_sc as plsc

_N = 10000
_E = 320000
_G = 64
_NC = 2
_NS = 16
_K = 64
_EP = 327680
_NCH1 = _EP // (_NC * _NS * _K)
_NCH2 = _EP // (_NS * _K)
_GRP = 40
_NBUF = 4
_NDUMP = 240
_NACC = _N + _NDUMP
_NPT = _NACC // _NS
_WC = 400
_NWC = _N // _WC
_ZR = 16
_DH = 128


@functools.cache
def _mesh():
    return plsc.VectorSubcoreMesh(core_axis_name="c", subcore_axis_name="s",
                                  num_cores=_NC, num_subcores=_NS)


def _zero_acc(zb_v, acc, s, D, sem):
    @pl.loop(0, _ZR)
    def _(r):
        @pl.loop(0, D // 16)
        def _(j):
            zb_v[r, pl.ds(j * 16, 16)] = jnp.zeros((16,), jnp.float32)

    for r in range(2):
        j = s + _NS * r

        @pl.when(j < _NWC)
        def _():
            @pl.loop(0, _WC // _ZR)
            def _(m):
                pltpu.async_copy(
                    zb_v, acc.at[pl.ds(j * _WC + m * _ZR, _ZR)], sem)

            @pl.loop(0, _WC // _ZR)
            def _(m):
                pltpu.make_async_copy(
                    zb_v, acc.at[pl.ds(j * _WC, _ZR)], sem).wait()


def _write_out(acc, dst_ref, s, sem):
    for r in range(2):
        j = s + _NS * r

        @pl.when(j < _NWC)
        def _():
            pltpu.async_copy(acc.at[pl.ds(j * _WC, _WC)],
                             dst_ref.at[pl.ds(j * _WC, _WC)], sem)

    for r in range(2):
        j = s + _NS * r

        @pl.when(j < _NWC)
        def _():
            pltpu.make_async_copy(acc.at[pl.ds(j * _WC, _WC)],
                                  dst_ref.at[pl.ds(j * _WC, _WC)], sem).wait()


def _agg_pipeline(idx_src, idx_dst, h_src, rows, acc,
                  src_v, dst_v, gsem, ssem, n_groups):
    @pl.loop(0, n_groups)
    def _(g):
        pltpu.sync_copy(idx_src.at[g], src_v)
        pltpu.sync_copy(idx_dst.at[g], dst_v)
        for b in range(_NBUF):
            pltpu.async_copy(h_src.at[src_v.at[b]], rows[b], gsem.at[b])

        @pl.loop(0, _GRP // _NBUF)
        def _(q):
            for b in range(_NBUF):
                ch = q * _NBUF + b
                pltpu.make_async_copy(
                    h_src.at[src_v.at[ch]], rows[b], gsem.at[b]).wait()
                pltpu.async_copy(
                    rows[b], acc.at[dst_v.at[ch]], ssem.at[b], add=True)
                nxt = ch + _NBUF

                @pl.when(nxt < _GRP)
                def _():
                    pltpu.make_async_copy(
                        rows[b], acc.at[dst_v.at[ch]], ssem.at[b]).wait()
                    pltpu.async_copy(
                        h_src.at[src_v.at[nxt]], rows[b], gsem.at[b])

        for b in range(_NBUF):
            pltpu.make_async_copy(
                rows[b], acc.at[dst_v.at[0]], ssem.at[b]).wait()


def _sc_degree(dst_t):
    cp = pltpu.CompilerParams()
    if "needs_layout_passes" in pltpu.CompilerParams.__dataclass_fields__:
        cp = dataclasses.replace(cp, needs_layout_passes=False)

    @functools.partial(
        pl.kernel,
        out_type=jax.ShapeDtypeStruct((_NC, _NS, _NPT // _DH, _DH),
                                      jnp.float32),
        compiler_params=cp,
        mesh=_mesh(),
        scratch_types=[
            pltpu.VMEM((_GRP, _K), jnp.int32),
            pltpu.VMEM((_NACC,), jnp.float32),
            pltpu.VMEM((_NS, _NPT), jnp.float32),
            pltpu.VMEM((_NPT // _DH, _DH), jnp.float32),
            pltpu.VMEM_SHARED((_NS, _NACC), jnp.float32),
        ],
    )
    def deg_kernel(dst_hbm, out_hbm, dst_v, hist, red, res, spm):
        c = lax.axis_index("c")
        s = lax.axis_index("s")
        ones = jnp.ones((16,), jnp.float32)

        @pl.loop(0, _NACC // 16)
        def _(i):
            hist[pl.ds(i * 16, 16)] = jnp.zeros((16,), jnp.float32)

        @pl.loop(0, _NCH1 // _GRP)
        def _(g):
            pltpu.sync_copy(dst_hbm.at[c, s, g], dst_v)

            @pl.loop(0, _GRP)
            def _(ch):
                for j in range(_K // 16):
                    vec = dst_v[ch, pl.ds(j * 16, 16)]
                    plsc.addupdate_scatter(hist, [vec], ones)

        pltpu.sync_copy(hist, spm.at[s])
        plsc.subcore_barrier()
        pltpu.sync_copy(spm.at[:, pl.ds(s * _NPT, _NPT)], red)
        for j in range(_NPT // 16):
            acc16 = red[0, pl.ds(j * 16, 16)]
            for r in range(1, _NS):
                acc16 = acc16 + red[r, pl.ds(j * 16, 16)]
            res[j // 8, pl.ds((j % 8) * 16, 16)] = acc16
        pltpu.sync_copy(res, out_hbm.at[c, s])

    return deg_kernel(dst_t)


def _agg_scratch(D):
    return [
        pltpu.VMEM((_GRP, _K), jnp.int32),
        pltpu.VMEM((_GRP, _K), jnp.int32),
    ] + [pltpu.VMEM((_K, D), jnp.float32) for _ in range(_NBUF)] + [
        pltpu.VMEM((_ZR, D), jnp.float32),
        pltpu.SemaphoreType.DMA((_NBUF,)),
        pltpu.SemaphoreType.DMA((_NBUF,)),
        pltpu.SemaphoreType.DMA,
        pltpu.VMEM_SHARED((_NACC, D), jnp.float32),
    ]


def _sc_agg_l1(h, src_t, dst_t):
    D = h.shape[1]

    @functools.partial(
        pl.kernel,
        out_type=jax.ShapeDtypeStruct((_NC, _N, D), jnp.float32),
        mesh=_mesh(),
        scratch_types=_agg_scratch(D),
    )
    def agg_kernel(h_hbm, src_hbm, dst_hbm, out_hbm,
                   src_v, dst_v, r0, r1, r2, r3, zb_v, gsem, ssem, msem, acc):
        c = lax.axis_index("c")
        s = lax.axis_index("s")
        _zero_acc(zb_v, acc, s, D, msem)
        plsc.subcore_barrier()
        _agg_pipeline(src_hbm.at[c, s], dst_hbm.at[c, s], h_hbm,
                      [r0, r1, r2, r3], acc, src_v, dst_v, gsem, ssem,
                      _NCH1 // _GRP)
        plsc.subcore_barrier()
        _write_out(acc, out_hbm.at[c], s, msem)

    return agg_kernel(h, src_t, dst_t)


def _sc_agg_cols(h2, src_t, dst_t):
    @functools.partial(
        pl.kernel,
        out_type=jax.ShapeDtypeStruct((_NC, _N, _DH), jnp.float32),
        mesh=_mesh(),
        scratch_types=_agg_scratch(_DH),
    )
    def agg_kernel(h_hbm, src_hbm, dst_hbm, out_hbm,
                   src_v, dst_v, r0, r1, r2, r3, zb_v, gsem, ssem, msem, acc):
        c = lax.axis_index("c")
        s = lax.axis_index("s")
        _zero_acc(zb_v, acc, s, _DH, msem)
        plsc.subcore_barrier()
        _agg_pipeline(src_hbm.at[s], dst_hbm.at[s], h_hbm.at[c],
                      [r0, r1, r2, r3], acc, src_v, dst_v, gsem, ssem,
                      _NCH2 // _GRP)
        plsc.subcore_barrier()
        _write_out(acc, out_hbm.at[c], s, msem)

    return agg_kernel(h2, src_t, dst_t)


_TCR = 2000


def _tc_dinv(degp):
    def body(dp_ref, out_ref):
        dp = dp_ref[...]
        out_ref[...] = lax.rsqrt(dp[0] + dp[1] + 1.0)

    return pl.pallas_call(
        body,
        out_shape=jax.ShapeDtypeStruct((_NACC // _DH, _DH), jnp.float32),
    )(degp)


def _tc_scale(dinv, x):
    F = x.shape[1]

    def body(dinv_ref, x_ref, xp_ref):
        xp_ref[...] = x_ref[...] * dinv_ref[...]

    return pl.pallas_call(
        body,
        grid=(_N // _TCR,),
        in_specs=[pl.BlockSpec((_TCR, 1), lambda i: (i, 0)),
                  pl.BlockSpec((_TCR, F), lambda i: (i, 0))],
        out_specs=pl.BlockSpec((_TCR, F), lambda i: (i, 0)),
        out_shape=jax.ShapeDtypeStruct((_N, F), jnp.float32),
    )(dinv, x)


def _split_cols(out_ref, t):
    out_ref[0] = t[:, :_DH]
    out_ref[1] = t[:, _DH:]


def _tc_layer1(p, xp, dinv, w1p, b1p, w2p):
    F = xp.shape[1]

    def body(p_ref, xp_ref, dinv_ref, w1_ref, b1_ref, w2_ref, out_ref):
        pp = p_ref[...]
        dinv = dinv_ref[...]
        g = dinv * (pp[0] + pp[1] + xp_ref[...])
        t = jnp.dot(g, w1_ref[...], preferred_element_type=jnp.float32)
        t = jnp.maximum(t + b1_ref[...], 0.0)
        t = dinv * jnp.dot(t, w2_ref[...], preferred_element_type=jnp.float32)
        _split_cols(out_ref, t)

    return pl.pallas_call(
        body,
        grid=(_N // _TCR,),
        in_specs=[pl.BlockSpec((2, _TCR, F), lambda i: (0, i, 0)),
                  pl.BlockSpec((_TCR, F), lambda i: (i, 0)),
                  pl.BlockSpec((_TCR, 1), lambda i: (i, 0)),
                  pl.BlockSpec(w1p.shape, lambda i: (0, 0)),
                  pl.BlockSpec(b1p.shape, lambda i: (0, 0)),
                  pl.BlockSpec(w2p.shape, lambda i: (0, 0))],
        out_specs=pl.BlockSpec((2, _TCR, _DH), lambda i: (0, i, 0)),
        out_shape=jax.ShapeDtypeStruct((_NC, _N, _DH), jnp.float32),
    )(p, xp, dinv, w1p, b1p, w2p)


def _merge_halves(p_ref, hp_ref):
    pp = p_ref[...]
    hh = hp_ref[...]
    return jnp.concatenate([pp[0] + hh[0], pp[1] + hh[1]], axis=1)


def _tc_mid(p, hp, dinv, bp, wp):
    def body(p_ref, hp_ref, dinv_ref, b_ref, w_ref, out_ref):
        dinv = dinv_ref[...]
        g = _merge_halves(p_ref, hp_ref)
        a = jnp.maximum(dinv * g + b_ref[...], 0.0)
        t = dinv * jnp.dot(a, w_ref[...], preferred_element_type=jnp.float32)
        _split_cols(out_ref, t)

    return pl.pallas_call(
        body,
        grid=(_N // _TCR,),
        in_specs=[pl.BlockSpec((2, _TCR, _DH), lambda i: (0, i, 0)),
                  pl.BlockSpec((2, _TCR, _DH), lambda i: (0, i, 0)),
                  pl.BlockSpec((_TCR, 1), lambda i: (i, 0)),
                  pl.BlockSpec(bp.shape, lambda i: (0, 0)),
                  pl.BlockSpec(wp.shape, lambda i: (0, 0))],
        out_specs=pl.BlockSpec((2, _TCR, _DH), lambda i: (0, i, 0)),
        out_shape=jax.ShapeDtypeStruct((_NC, _N, _DH), jnp.float32),
    )(p, hp, dinv, bp, wp)


def _tc_last(p, hp, dinv, bp):
    D = 2 * _DH

    def body(p_ref, hp_ref, dinv_ref, b_ref, out_ref):
        dinv = dinv_ref[...]
        g = _merge_halves(p_ref, hp_ref)
        out_ref[...] = jnp.maximum(dinv * g + b_ref[...], 0.0)

    return pl.pallas_call(
        body,
        grid=(_N // _TCR,),
        in_specs=[pl.BlockSpec((2, _TCR, _DH), lambda i: (0, i, 0)),
                  pl.BlockSpec((2, _TCR, _DH), lambda i: (0, i, 0)),
                  pl.BlockSpec((_TCR, 1), lambda i: (i, 0)),
                  pl.BlockSpec(bp.shape, lambda i: (0, 0))],
        out_specs=pl.BlockSpec((_TCR, D), lambda i: (i, 0)),
        out_shape=jax.ShapeDtypeStruct((_N, D), jnp.float32),
    )(p, hp, dinv, bp)


def _tc_pool_head(a4, batch_row, fc1w, fc1b2, fc2w, fc2b2, hdim):
    D = a4.shape[1]
    NEG = -3.0e38

    def body(a4_ref, bt_ref, w1_ref, b1_ref, w2_ref, b2_ref, out_ref, max_s):
        a = a4_ref[...]
        btr = bt_ref[...]
        gi = lax.broadcasted_iota(jnp.int32, (_G, _N), 0)
        oh = (btr == gi).astype(jnp.float32)
        gsum = jnp.dot(oh, a, preferred_element_type=jnp.float32)
        cnt = jnp.sum(oh, axis=1, keepdims=True)

        bm = jnp.max(a.reshape(_N // 8, 8, D), axis=1)
        gidc = lax.broadcasted_iota(jnp.int32, (_G, 1), 0)
        ki = lax.broadcasted_iota(jnp.int32, (_N // 8, 1), 0)
        ri = lax.broadcasted_iota(jnp.int32, (8, 1), 0)

        def seg(g, carry):
            e0 = jnp.sum(jnp.where(gidc <= g, cnt, 0.0)).astype(jnp.int32)
            n0 = jnp.sum(jnp.where(gidc == g, cnt, 0.0)).astype(jnp.int32)
            s0 = e0 - n0
            kb0 = (s0 + 7) // 8
            kb1 = e0 // 8
            m = jnp.max(jnp.where((ki >= kb0) & (ki < kb1), bm, NEG),
                        axis=0, keepdims=True)
            hb = jnp.clip(s0 // 8, 0, _N // 8 - 1)
            tb = jnp.clip(kb1, 0, _N // 8 - 1)
            hrows = a4_ref[pl.ds(hb * 8, 8), :]
            hmask = (ri + hb * 8 >= s0) & (ri + hb * 8 < e0)
            m = jnp.maximum(m, jnp.max(jnp.where(hmask, hrows, NEG),
                                       axis=0, keepdims=True))
            trows = a4_ref[pl.ds(tb * 8, 8), :]
            tmask = (ri + tb * 8 >= s0) & (ri + tb * 8 < e0)
            m = jnp.maximum(m, jnp.max(jnp.where(tmask, trows, NEG),
                                       axis=0, keepdims=True))
            max_s[pl.ds(g, 1), :] = m
            return carry

        lax.fori_loop(0, _G, seg, 0)
        gmaxv = max_s[...]
        gmean = gsum / jnp.maximum(cnt, 1.0)
        gmax = jnp.where(cnt > 0.0, gmaxv, 0.0)
        z = jnp.concatenate([gmean[:, :hdim], gmax[:, :hdim]], axis=1)
        z = jnp.dot(z, w1_ref[...], preferred_element_type=jnp.float32)
        z = jnp.maximum(z + b1_ref[...], 0.0)
        logits = jnp.dot(z, w2_ref[...], preferred_element_type=jnp.float32)
        logits = logits + b2_ref[...]
        mx = jnp.max(logits, axis=1, keepdims=True)
        lse = jnp.log(jnp.sum(jnp.exp(logits - mx), axis=1, keepdims=True)) + mx
        out_ref[...] = logits - lse

    return pl.pallas_call(
        body,
        out_shape=jax.ShapeDtypeStruct((_G, fc2w.shape[1]), jnp.float32),
        scratch_shapes=[pltpu.VMEM((_G, D), jnp.float32)],
    )(a4, batch_row, fc1w, fc1b2, fc2w, fc2b2)


def kernel(x, edge_index, batch, W1, b1, W2, b2, W3, b3, W4, b4,
           fc1W, fc1b, fc2W, fc2b):
    H = W1.shape[1]
    D = 2 * _DH

    npad = _EP - _E
    d_src = (jnp.arange(npad, dtype=edge_index.dtype) * 61) % _N
    d_dst = _N + jnp.arange(npad, dtype=edge_index.dtype) % _NDUMP

    def _tiled(v, dv):
        nt = _NC * _NS
        return jnp.concatenate(
            [v.reshape(nt, _E // nt), dv.reshape(nt, npad // nt)],
            axis=1).reshape(-1)

    src_pad = _tiled(edge_index[0], d_src)
    dst_pad = _tiled(edge_index[1], d_dst)
    src1 = src_pad.reshape(_NC, _NS, _NCH1 // _GRP, _GRP, _K)
    dst1 = dst_pad.reshape(_NC, _NS, _NCH1 // _GRP, _GRP, _K)
    src2 = src_pad.reshape(_NS, _NCH2 // _GRP, _GRP, _K)
    dst2 = dst_pad.reshape(_NS, _NCH2 // _GRP, _GRP, _K)
    batch_row = batch.reshape(1, _N)

    pad_c = D - H
    w1p = jnp.pad(W1, ((0, 0), (0, pad_c)))
    w2p = jnp.pad(W2, ((0, pad_c), (0, pad_c)))
    w3p = jnp.pad(W3, ((0, pad_c), (0, pad_c)))
    w4p = jnp.pad(W4, ((0, pad_c), (0, pad_c)))
    b1p = jnp.pad(b1, (0, pad_c)).reshape(1, D)
    b2p = jnp.pad(b2, (0, pad_c)).reshape(1, D)
    b3p = jnp.pad(b3, (0, pad_c)).reshape(1, D)
    b4p = jnp.pad(b4, (0, pad_c)).reshape(1, D)
    fc1b2 = fc1b.reshape(1, -1)
    fc2b2 = fc2b.reshape(1, -1)

    degp = _sc_degree(dst1).reshape(_NC, _NACC // _DH, _DH)
    dinvl = _tc_dinv(degp)
    dinv = dinvl.reshape(_NACC, 1)[:_N]
    xp = _tc_scale(dinv, x)
    p1 = _sc_agg_l1(xp, src1, dst1)
    h2p = _tc_layer1(p1, xp, dinv, w1p, b1p, w2p)
    p2 = _sc_agg_cols(h2p, src2, dst2)
    h3p = _tc_mid(p2, h2p, dinv, b2p, w3p)
    p3 = _sc_agg_cols(h3p, src2, dst2)
    h4p = _tc_mid(p3, h3p, dinv, b3p, w4p)
    p4 = _sc_agg_cols(h4p, src2, dst2)
    a4 = _tc_last(p4, h4p, dinv, b4p)
    return _tc_pool_head(a4, batch_row, fc1W, fc1b2, fc2W, fc2b2, H)


# speedup vs baseline: 1.1390x; 1.0745x over previous
import dataclasses
import functools

import jax
import jax.numpy as jnp
from jax import lax
from jax.experimental import pallas as pl
from jax.experimental.pallas import tpu as pltpu
from jax.experimental.pallas import tpu_sc as plsc

_N = 10000
_E = 320000
_G = 64
_NC = 2
_NS = 16
_K = 64
_EP = 327680
_NCH1 = _EP // (_NC * _NS * _K)
_NCH2 = _EP // (_NS * _K)
_GRP = 20
_NBUF = 4
_NDUMP = 240
_NACC = _N + _NDUMP
_NPT = _NACC // _NS
_WC = 400
_NWC = _N // _WC
_ZR = 16
_DH = 128


@functools.cache
def _mesh():
    return plsc.VectorSubcoreMesh(core_axis_name="c", subcore_axis_name="s",
                                  num_cores=_NC, num_subcores=_NS)


def _zero_acc(zb_v, acc, s, D, sem):
    @pl.loop(0, _ZR)
    def _(r):
        @pl.loop(0, D // 16)
        def _(j):
            zb_v[r, pl.ds(j * 16, 16)] = jnp.zeros((16,), jnp.float32)

    for r in range(2):
        j = s + _NS * r

        @pl.when(j < _NWC)
        def _():
            @pl.loop(0, _WC // _ZR)
            def _(m):
                pltpu.async_copy(
                    zb_v, acc.at[pl.ds(j * _WC + m * _ZR, _ZR)], sem)

            @pl.loop(0, _WC // _ZR)
            def _(m):
                pltpu.make_async_copy(
                    zb_v, acc.at[pl.ds(j * _WC, _ZR)], sem).wait()


def _write_out(acc, dst_ref, s, sem):
    for r in range(2):
        j = s + _NS * r

        @pl.when(j < _NWC)
        def _():
            pltpu.async_copy(acc.at[pl.ds(j * _WC, _WC)],
                             dst_ref.at[pl.ds(j * _WC, _WC)], sem)

    for r in range(2):
        j = s + _NS * r

        @pl.when(j < _NWC)
        def _():
            pltpu.make_async_copy(acc.at[pl.ds(j * _WC, _WC)],
                                  dst_ref.at[pl.ds(j * _WC, _WC)], sem).wait()


def _agg_pipeline(idx_src, idx_dst, h_src, rows, acc,
                  sva, dva, svb, dvb, gsem, ssem, isem, n_groups):
    QN = _GRP // _NBUF

    pltpu.sync_copy(idx_src.at[0], sva)
    pltpu.sync_copy(idx_dst.at[0], dva)
    for b in range(_NBUF):
        pltpu.async_copy(h_src.at[sva.at[b]], rows[b], gsem.at[b])

    def half(sv, dv, nsv, ndv, pf_idx):
        for b in range(_NBUF):
            pltpu.make_async_copy(
                h_src.at[sv.at[b]], rows[b], gsem.at[b]).wait()
            pltpu.async_copy(rows[b], acc.at[dv.at[b]], ssem.at[b], add=True)
            pltpu.make_async_copy(
                rows[b], acc.at[dv.at[b]], ssem.at[b]).wait()
            pltpu.async_copy(
                h_src.at[sv.at[b + _NBUF]], rows[b], gsem.at[b])

        @pl.when(pf_idx < n_groups)
        def _():
            pltpu.async_copy(idx_src.at[pf_idx], nsv, isem)
            pltpu.async_copy(idx_dst.at[pf_idx], ndv, isem)

        @pl.loop(1, QN)
        def _(q):
            for b in range(_NBUF):
                ch = q * _NBUF + b
                pltpu.make_async_copy(
                    h_src.at[sv.at[ch]], rows[b], gsem.at[b]).wait()
                pltpu.async_copy(
                    rows[b], acc.at[dv.at[ch]], ssem.at[b], add=True)
                pltpu.make_async_copy(
                    rows[b], acc.at[dv.at[ch]], ssem.at[b]).wait()

                @pl.when(q < QN - 1)
                def _():
                    pltpu.async_copy(
                        h_src.at[sv.at[ch + _NBUF]], rows[b], gsem.at[b])

                @pl.when((q == QN - 1) & (pf_idx < n_groups))
                def _():
                    if b == 0:
                        pltpu.make_async_copy(
                            idx_src.at[0], nsv, isem).wait()
                        pltpu.make_async_copy(
                            idx_dst.at[0], ndv, isem).wait()
                    pltpu.async_copy(
                        h_src.at[nsv.at[b]], rows[b], gsem.at[b])

    @pl.loop(0, n_groups // 2)
    def _(gg):
        half(sva, dva, svb, dvb, 2 * gg + 1)
        half(svb, dvb, sva, dva, 2 * gg + 2)


def _sc_degree(dst_t):
    cp = pltpu.CompilerParams()
    if "needs_layout_passes" in pltpu.CompilerParams.__dataclass_fields__:
        cp = dataclasses.replace(cp, needs_layout_passes=False)

    @functools.partial(
        pl.kernel,
        out_type=jax.ShapeDtypeStruct((_NC, _NS, _NPT // _DH, _DH),
                                      jnp.float32),
        compiler_params=cp,
        mesh=_mesh(),
        scratch_types=[
            pltpu.VMEM((_GRP, _K), jnp.int32),
            pltpu.VMEM((_NACC,), jnp.float32),
            pltpu.VMEM((_NS, _NPT), jnp.float32),
            pltpu.VMEM((_NPT // _DH, _DH), jnp.float32),
            pltpu.VMEM_SHARED((_NS, _NACC), jnp.float32),
        ],
    )
    def deg_kernel(dst_hbm, out_hbm, dst_v, hist, red, res, spm):
        c = lax.axis_index("c")
        s = lax.axis_index("s")
        ones = jnp.ones((16,), jnp.float32)

        @pl.loop(0, _NACC // 16)
        def _(i):
            hist[pl.ds(i * 16, 16)] = jnp.zeros((16,), jnp.float32)

        @pl.loop(0, _NCH1 // _GRP)
        def _(g):
            pltpu.sync_copy(dst_hbm.at[c, s, g], dst_v)

            @pl.loop(0, _GRP)
            def _(ch):
                for j in range(_K // 16):
                    vec = dst_v[ch, pl.ds(j * 16, 16)]
                    plsc.addupdate_scatter(hist, [vec], ones)

        pltpu.sync_copy(hist, spm.at[s])
        plsc.subcore_barrier()
        pltpu.sync_copy(spm.at[:, pl.ds(s * _NPT, _NPT)], red)
        for j in range(_NPT // 16):
            acc16 = red[0, pl.ds(j * 16, 16)]
            for r in range(1, _NS):
                acc16 = acc16 + red[r, pl.ds(j * 16, 16)]
            res[j // 8, pl.ds((j % 8) * 16, 16)] = acc16
        pltpu.sync_copy(res, out_hbm.at[c, s])

    return deg_kernel(dst_t)


def _agg_scratch(D):
    return [
        pltpu.VMEM((_GRP, _K), jnp.int32),
        pltpu.VMEM((_GRP, _K), jnp.int32),
        pltpu.VMEM((_GRP, _K), jnp.int32),
        pltpu.VMEM((_GRP, _K), jnp.int32),
    ] + [pltpu.VMEM((_K, D), jnp.float32) for _ in range(_NBUF)] + [
        pltpu.VMEM((_ZR, D), jnp.float32),
        pltpu.SemaphoreType.DMA((_NBUF,)),
        pltpu.SemaphoreType.DMA((_NBUF,)),
        pltpu.SemaphoreType.DMA,
        pltpu.SemaphoreType.DMA,
        pltpu.VMEM_SHARED((_NACC, D), jnp.float32),
    ]


def _sc_agg_l1(h, src_t, dst_t):
    D = h.shape[1]

    @functools.partial(
        pl.kernel,
        out_type=jax.ShapeDtypeStruct((_NC, _N, D), jnp.float32),
        mesh=_mesh(),
        scratch_types=_agg_scratch(D),
    )
    def agg_kernel(h_hbm, src_hbm, dst_hbm, out_hbm,
                   sva, dva, svb, dvb, r0, r1, r2, r3, zb_v,
                   gsem, ssem, msem, isem, acc):
        c = lax.axis_index("c")
        s = lax.axis_index("s")
        _zero_acc(zb_v, acc, s, D, msem)
        plsc.subcore_barrier()
        _agg_pipeline(src_hbm.at[c, s], dst_hbm.at[c, s], h_hbm,
                      [r0, r1, r2, r3], acc, sva, dva, svb, dvb,
                      gsem, ssem, isem, _NCH1 // _GRP)
        plsc.subcore_barrier()
        _write_out(acc, out_hbm.at[c], s, msem)

    return agg_kernel(h, src_t, dst_t)


def _sc_agg_cols(h2, src_t, dst_t):
    @functools.partial(
        pl.kernel,
        out_type=jax.ShapeDtypeStruct((_NC, _N, _DH), jnp.float32),
        mesh=_mesh(),
        scratch_types=_agg_scratch(_DH),
    )
    def agg_kernel(h_hbm, src_hbm, dst_hbm, out_hbm,
                   sva, dva, svb, dvb, r0, r1, r2, r3, zb_v,
                   gsem, ssem, msem, isem, acc):
        c = lax.axis_index("c")
        s = lax.axis_index("s")
        _zero_acc(zb_v, acc, s, _DH, msem)
        plsc.subcore_barrier()
        _agg_pipeline(src_hbm.at[s], dst_hbm.at[s], h_hbm.at[c],
                      [r0, r1, r2, r3], acc, sva, dva, svb, dvb,
                      gsem, ssem, isem, _NCH2 // _GRP)
        plsc.subcore_barrier()
        _write_out(acc, out_hbm.at[c], s, msem)

    return agg_kernel(h2, src_t, dst_t)


_TCR = 2000


def _tc_dinv(degp):
    def body(dp_ref, out_ref):
        dp = dp_ref[...]
        out_ref[...] = lax.rsqrt(dp[0] + dp[1] + 1.0)

    return pl.pallas_call(
        body,
        out_shape=jax.ShapeDtypeStruct((_NACC // _DH, _DH), jnp.float32),
    )(degp)


def _tc_scale(dinv, x):
    F = x.shape[1]

    def body(dinv_ref, x_ref, xp_ref):
        xp_ref[...] = x_ref[...] * dinv_ref[...]

    return pl.pallas_call(
        body,
        grid=(_N // _TCR,),
        in_specs=[pl.BlockSpec((_TCR, 1), lambda i: (i, 0)),
                  pl.BlockSpec((_TCR, F), lambda i: (i, 0))],
        out_specs=pl.BlockSpec((_TCR, F), lambda i: (i, 0)),
        out_shape=jax.ShapeDtypeStruct((_N, F), jnp.float32),
    )(dinv, x)


def _split_cols(out_ref, t):
    out_ref[0] = t[:, :_DH]
    out_ref[1] = t[:, _DH:]


def _tc_layer1(p, xp, dinv, w1p, b1p, w2p):
    F = xp.shape[1]

    def body(p_ref, xp_ref, dinv_ref, w1_ref, b1_ref, w2_ref, out_ref):
        pp = p_ref[...]
        dinv = dinv_ref[...]
        g = dinv * (pp[0] + pp[1] + xp_ref[...])
        t = jnp.dot(g, w1_ref[...], preferred_element_type=jnp.float32)
        t = jnp.maximum(t + b1_ref[...], 0.0)
        t = dinv * jnp.dot(t, w2_ref[...], preferred_element_type=jnp.float32)
        _split_cols(out_ref, t)

    return pl.pallas_call(
        body,
        grid=(_N // _TCR,),
        in_specs=[pl.BlockSpec((2, _TCR, F), lambda i: (0, i, 0)),
                  pl.BlockSpec((_TCR, F), lambda i: (i, 0)),
                  pl.BlockSpec((_TCR, 1), lambda i: (i, 0)),
                  pl.BlockSpec(w1p.shape, lambda i: (0, 0)),
                  pl.BlockSpec(b1p.shape, lambda i: (0, 0)),
                  pl.BlockSpec(w2p.shape, lambda i: (0, 0))],
        out_specs=pl.BlockSpec((2, _TCR, _DH), lambda i: (0, i, 0)),
        out_shape=jax.ShapeDtypeStruct((_NC, _N, _DH), jnp.float32),
    )(p, xp, dinv, w1p, b1p, w2p)


def _merge_halves(p_ref, hp_ref):
    pp = p_ref[...]
    hh = hp_ref[...]
    return jnp.concatenate([pp[0] + hh[0], pp[1] + hh[1]], axis=1)


def _tc_mid(p, hp, dinv, bp, wp):
    def body(p_ref, hp_ref, dinv_ref, b_ref, w_ref, out_ref):
        dinv = dinv_ref[...]
        g = _merge_halves(p_ref, hp_ref)
        a = jnp.maximum(dinv * g + b_ref[...], 0.0)
        t = dinv * jnp.dot(a, w_ref[...], preferred_element_type=jnp.float32)
        _split_cols(out_ref, t)

    return pl.pallas_call(
        body,
        grid=(_N // _TCR,),
        in_specs=[pl.BlockSpec((2, _TCR, _DH), lambda i: (0, i, 0)),
                  pl.BlockSpec((2, _TCR, _DH), lambda i: (0, i, 0)),
                  pl.BlockSpec((_TCR, 1), lambda i: (i, 0)),
                  pl.BlockSpec(bp.shape, lambda i: (0, 0)),
                  pl.BlockSpec(wp.shape, lambda i: (0, 0))],
        out_specs=pl.BlockSpec((2, _TCR, _DH), lambda i: (0, i, 0)),
        out_shape=jax.ShapeDtypeStruct((_NC, _N, _DH), jnp.float32),
    )(p, hp, dinv, bp, wp)


def _tc_last(p, hp, dinv, bp):
    D = 2 * _DH

    def body(p_ref, hp_ref, dinv_ref, b_ref, out_ref):
        dinv = dinv_ref[...]
        g = _merge_halves(p_ref, hp_ref)
        out_ref[...] = jnp.maximum(dinv * g + b_ref[...], 0.0)

    return pl.pallas_call(
        body,
        grid=(_N // _TCR,),
        in_specs=[pl.BlockSpec((2, _TCR, _DH), lambda i: (0, i, 0)),
                  pl.BlockSpec((2, _TCR, _DH), lambda i: (0, i, 0)),
                  pl.BlockSpec((_TCR, 1), lambda i: (i, 0)),
                  pl.BlockSpec(bp.shape, lambda i: (0, 0))],
        out_specs=pl.BlockSpec((_TCR, D), lambda i: (i, 0)),
        out_shape=jax.ShapeDtypeStruct((_N, D), jnp.float32),
    )(p, hp, dinv, bp)


def _tc_pool_head(a4, batch_row, fc1w, fc1b2, fc2w, fc2b2, hdim):
    D = a4.shape[1]
    NEG = -3.0e38

    def body(a4_ref, bt_ref, w1_ref, b1_ref, w2_ref, b2_ref, out_ref, max_s):
        a = a4_ref[...]
        btr = bt_ref[...]
        gi = lax.broadcasted_iota(jnp.int32, (_G, _N), 0)
        oh = (btr == gi).astype(jnp.float32)
        gsum = jnp.dot(oh, a, preferred_element_type=jnp.float32)
        cnt = jnp.sum(oh, axis=1, keepdims=True)

        bm = jnp.max(a.reshape(_N // 8, 8, D), axis=1)
        gidc = lax.broadcasted_iota(jnp.int32, (_G, 1), 0)
        ki = lax.broadcasted_iota(jnp.int32, (_N // 8, 1), 0)
        ri = lax.broadcasted_iota(jnp.int32, (8, 1), 0)

        def seg(g, carry):
            e0 = jnp.sum(jnp.where(gidc <= g, cnt, 0.0)).astype(jnp.int32)
            n0 = jnp.sum(jnp.where(gidc == g, cnt, 0.0)).astype(jnp.int32)
            s0 = e0 - n0
            kb0 = (s0 + 7) // 8
            kb1 = e0 // 8
            m = jnp.max(jnp.where((ki >= kb0) & (ki < kb1), bm, NEG),
                        axis=0, keepdims=True)
            hb = jnp.clip(s0 // 8, 0, _N // 8 - 1)
            tb = jnp.clip(kb1, 0, _N // 8 - 1)
            hrows = a4_ref[pl.ds(hb * 8, 8), :]
            hmask = (ri + hb * 8 >= s0) & (ri + hb * 8 < e0)
            m = jnp.maximum(m, jnp.max(jnp.where(hmask, hrows, NEG),
                                       axis=0, keepdims=True))
            trows = a4_ref[pl.ds(tb * 8, 8), :]
            tmask = (ri + tb * 8 >= s0) & (ri + tb * 8 < e0)
            m = jnp.maximum(m, jnp.max(jnp.where(tmask, trows, NEG),
                                       axis=0, keepdims=True))
            max_s[pl.ds(g, 1), :] = m
            return carry

        lax.fori_loop(0, _G, seg, 0)
        gmaxv = max_s[...]
        gmean = gsum / jnp.maximum(cnt, 1.0)
        gmax = jnp.where(cnt > 0.0, gmaxv, 0.0)
        z = jnp.concatenate([gmean[:, :hdim], gmax[:, :hdim]], axis=1)
        z = jnp.dot(z, w1_ref[...], preferred_element_type=jnp.float32)
        z = jnp.maximum(z + b1_ref[...], 0.0)
        logits = jnp.dot(z, w2_ref[...], preferred_element_type=jnp.float32)
        logits = logits + b2_ref[...]
        mx = jnp.max(logits, axis=1, keepdims=True)
        lse = jnp.log(jnp.sum(jnp.exp(logits - mx), axis=1, keepdims=True)) + mx
        out_ref[...] = logits - lse

    return pl.pallas_call(
        body,
        out_shape=jax.ShapeDtypeStruct((_G, fc2w.shape[1]), jnp.float32),
        scratch_shapes=[pltpu.VMEM((_G, D), jnp.float32)],
    )(a4, batch_row, fc1w, fc1b2, fc2w, fc2b2)


def kernel(x, edge_index, batch, W1, b1, W2, b2, W3, b3, W4, b4,
           fc1W, fc1b, fc2W, fc2b):
    H = W1.shape[1]
    D = 2 * _DH

    npad = _EP - _E
    d_src = (jnp.arange(npad, dtype=edge_index.dtype) * 61) % _N
    d_dst = _N + jnp.arange(npad, dtype=edge_index.dtype) % _NDUMP

    def _tiled(v, dv):
        nt = _NC * _NS
        return jnp.concatenate(
            [v.reshape(nt, _E // nt), dv.reshape(nt, npad // nt)],
            axis=1).reshape(-1)

    src_pad = _tiled(edge_index[0], d_src)
    dst_pad = _tiled(edge_index[1], d_dst)
    src1 = src_pad.reshape(_NC, _NS, _NCH1 // _GRP, _GRP, _K)
    dst1 = dst_pad.reshape(_NC, _NS, _NCH1 // _GRP, _GRP, _K)
    src2 = src_pad.reshape(_NS, _NCH2 // _GRP, _GRP, _K)
    dst2 = dst_pad.reshape(_NS, _NCH2 // _GRP, _GRP, _K)
    batch_row = batch.reshape(1, _N)

    pad_c = D - H
    w1p = jnp.pad(W1, ((0, 0), (0, pad_c)))
    w2p = jnp.pad(W2, ((0, pad_c), (0, pad_c)))
    w3p = jnp.pad(W3, ((0, pad_c), (0, pad_c)))
    w4p = jnp.pad(W4, ((0, pad_c), (0, pad_c)))
    b1p = jnp.pad(b1, (0, pad_c)).reshape(1, D)
    b2p = jnp.pad(b2, (0, pad_c)).reshape(1, D)
    b3p = jnp.pad(b3, (0, pad_c)).reshape(1, D)
    b4p = jnp.pad(b4, (0, pad_c)).reshape(1, D)
    fc1b2 = fc1b.reshape(1, -1)
    fc2b2 = fc2b.reshape(1, -1)

    degp = _sc_degree(dst1).reshape(_NC, _NACC // _DH, _DH)
    dinvl = _tc_dinv(degp)
    dinv = dinvl.reshape(_NACC, 1)[:_N]
    xp = _tc_scale(dinv, x)
    p1 = _sc_agg_l1(xp, src1, dst1)
    h2p = _tc_layer1(p1, xp, dinv, w1p, b1p, w2p)
    p2 = _sc_agg_cols(h2p, src2, dst2)
    h3p = _tc_mid(p2, h2p, dinv, b2p, w3p)
    p3 = _sc_agg_cols(h3p, src2, dst2)
    h4p = _tc_mid(p3, h3p, dinv, b3p, w4p)
    p4 = _sc_agg_cols(h4p, src2, dst2)
    a4 = _tc_last(p4, h4p, dinv, b4p)
    return _tc_pool_head(a4, batch_row, fc1W, fc1b2, fc2W, fc2b2, H)
